# baseline (speedup 1.0000x reference)
import jax
import jax.numpy as jnp
from jax import lax
from jax.experimental import pallas as pl
from jax.experimental.pallas import tpu as pltpu

_CH = 128


def _fused(table, idx0, idx1, W, b2d, seq_len):
    B = idx0.shape[0]
    D = table.shape[1]
    nch = B // _CH

    def body(i0_ref, i1_ref, table_ref, w_hbm, b_ref, o_ref,
             x_v, w_v, wsem, csems):
        pltpu.make_async_copy(w_hbm, w_v, wsem).start()

        def issue_row(r, _):
            flat = i0_ref[r] * seq_len + i1_ref[r]
            pltpu.make_async_copy(
                table_ref.at[flat], x_v.at[r], csems.at[r // _CH]
            ).start()
            return _

        lax.fori_loop(0, B, issue_row, 0, unroll=8)

        pltpu.make_async_copy(w_hbm, w_v, wsem).wait()
        for c in range(nch):
            sl = pl.ds(c * _CH, _CH)
            pltpu.make_async_copy(
                table_ref.at[pl.ds(0, _CH)], x_v.at[sl], csems.at[c]
            ).wait()
            acc = lax.dot_general(
                x_v[sl, :], w_v[...],
                (((1,), (1,)), ((), ())),
                preferred_element_type=jnp.float32,
            )
            o_ref[sl, :] = jnp.tanh(acc + b_ref[...])

    return pl.pallas_call(
        body,
        in_specs=[
            pl.BlockSpec(memory_space=pltpu.SMEM),
            pl.BlockSpec(memory_space=pltpu.SMEM),
            pl.BlockSpec(memory_space=pltpu.HBM),
            pl.BlockSpec(memory_space=pltpu.HBM),
            pl.BlockSpec(memory_space=pltpu.VMEM),
        ],
        out_specs=pl.BlockSpec(memory_space=pltpu.VMEM),
        out_shape=jax.ShapeDtypeStruct((B, D), jnp.float32),
        scratch_shapes=[
            pltpu.VMEM((B, D), jnp.float32),
            pltpu.VMEM((D, D), jnp.float32),
            pltpu.SemaphoreType.DMA,
            pltpu.SemaphoreType.DMA((nch,)),
        ],
    )(idx0, idx1, table, W, b2d)


def kernel(hidden_states, cls_indexes, W, b):
    n_batch, seq_len, D = hidden_states.shape
    table = hidden_states.reshape(n_batch * seq_len, D)
    idx = cls_indexes.astype(jnp.int32)
    return _fused(table, idx[:, 0], idx[:, 1], W, b.reshape(1, D), seq_len)


# speedup vs baseline: 1.1833x; 1.1833x over previous
import jax
import jax.numpy as jnp
from jax import lax
from jax.experimental import pallas as pl
from jax.experimental.pallas import tpu as pltpu

_CH = 256


def _fused(table, idx0, idx1, W, b2d, seq_len):
    B = idx0.shape[0]
    D = table.shape[1]
    nch = B // _CH

    def body(i0_ref, i1_ref, table_ref, w_hbm, b_ref, o_ref,
             x_v, w_v, wsem, csems):
        pltpu.make_async_copy(w_hbm, w_v, wsem).start()

        def issue_row(r, _):
            flat = i0_ref[r] * seq_len + i1_ref[r]
            pltpu.make_async_copy(
                table_ref.at[flat], x_v.at[r], csems.at[r // _CH]
            ).start()
            return _

        lax.fori_loop(0, B, issue_row, 0, unroll=16)

        pltpu.make_async_copy(w_hbm, w_v, wsem).wait()
        for c in range(nch):
            sl = pl.ds(c * _CH, _CH)
            pltpu.make_async_copy(
                table_ref.at[pl.ds(0, _CH)], x_v.at[sl], csems.at[c]
            ).wait()
            acc = lax.dot_general(
                x_v[sl, :], w_v[...],
                (((1,), (1,)), ((), ())),
                preferred_element_type=jnp.float32,
            )
            o_ref[sl, :] = jnp.tanh(acc + b_ref[...])

    return pl.pallas_call(
        body,
        in_specs=[
            pl.BlockSpec(memory_space=pltpu.SMEM),
            pl.BlockSpec(memory_space=pltpu.SMEM),
            pl.BlockSpec(memory_space=pltpu.HBM),
            pl.BlockSpec(memory_space=pltpu.HBM),
            pl.BlockSpec(memory_space=pltpu.VMEM),
        ],
        out_specs=pl.BlockSpec(memory_space=pltpu.VMEM),
        out_shape=jax.ShapeDtypeStruct((B, D), jnp.float32),
        scratch_shapes=[
            pltpu.VMEM((B, D), jnp.float32),
            pltpu.VMEM((D, D), jnp.float32),
            pltpu.SemaphoreType.DMA,
            pltpu.SemaphoreType.DMA((nch,)),
        ],
    )(idx0, idx1, table, W, b2d)


def kernel(hidden_states, cls_indexes, W, b):
    n_batch, seq_len, D = hidden_states.shape
    table = hidden_states.reshape(n_batch * seq_len, D)
    idx = cls_indexes.astype(jnp.int32)
    return _fused(table, idx[:, 0], idx[:, 1], W, b.reshape(1, D), seq_len)


# speedup vs baseline: 1.2278x; 1.0376x over previous
import jax
import jax.numpy as jnp
from jax import lax
from jax.experimental import pallas as pl
from jax.experimental.pallas import tpu as pltpu

_CH = 256


def _fused(table, idx0, idx1, W, b2d, seq_len):
    B = idx0.shape[0]
    D = table.shape[1]
    nch = B // _CH

    def body(i0_ref, i1_ref, table_ref, w_hbm, b_ref, o_hbm,
             x_v, w_v, o_v, wsem, csems, osems):
        pltpu.make_async_copy(w_hbm, w_v, wsem).start()

        def issue_row(r, _):
            flat = i0_ref[r] * seq_len + i1_ref[r]
            pltpu.make_async_copy(
                table_ref.at[flat], x_v.at[r], csems.at[r // _CH]
            ).start()
            return _

        lax.fori_loop(0, B, issue_row, 0, unroll=16)

        pltpu.make_async_copy(w_hbm, w_v, wsem).wait()
        for c in range(nch):
            sl = pl.ds(c * _CH, _CH)
            pltpu.make_async_copy(
                table_ref.at[pl.ds(0, _CH)], x_v.at[sl], csems.at[c]
            ).wait()
            acc = lax.dot_general(
                x_v[sl, :], w_v[...],
                (((1,), (1,)), ((), ())),
                preferred_element_type=jnp.float32,
            )
            o_v[c] = jnp.tanh(acc + b_ref[...])
            pltpu.make_async_copy(o_v.at[c], o_hbm.at[sl], osems.at[c]).start()
        for c in range(nch):
            pltpu.make_async_copy(o_v.at[c], o_hbm.at[pl.ds(c * _CH, _CH)],
                                  osems.at[c]).wait()

    return pl.pallas_call(
        body,
        in_specs=[
            pl.BlockSpec(memory_space=pltpu.SMEM),
            pl.BlockSpec(memory_space=pltpu.SMEM),
            pl.BlockSpec(memory_space=pltpu.HBM),
            pl.BlockSpec(memory_space=pltpu.HBM),
            pl.BlockSpec(memory_space=pltpu.VMEM),
        ],
        out_specs=pl.BlockSpec(memory_space=pltpu.HBM),
        out_shape=jax.ShapeDtypeStruct((B, D), jnp.float32),
        scratch_shapes=[
            pltpu.VMEM((B, D), jnp.float32),
            pltpu.VMEM((D, D), jnp.float32),
            pltpu.VMEM((nch, _CH, D), jnp.float32),
            pltpu.SemaphoreType.DMA,
            pltpu.SemaphoreType.DMA((nch,)),
            pltpu.SemaphoreType.DMA((nch,)),
        ],
    )(idx0, idx1, table, W, b2d)


def kernel(hidden_states, cls_indexes, W, b):
    n_batch, seq_len, D = hidden_states.shape
    table = hidden_states.reshape(n_batch * seq_len, D)
    idx = cls_indexes.astype(jnp.int32)
    return _fused(table, idx[:, 0], idx[:, 1], W, b.reshape(1, D), seq_len)


# speedup vs baseline: 1.4464x; 1.1781x over previous
import jax
import jax.numpy as jnp
from jax import lax
from jax.experimental import pallas as pl
from jax.experimental.pallas import tpu as pltpu

_CH = 256
_WH = 512


def _fused(table, idx0, idx1, W, b2d, seq_len):
    B = idx0.shape[0]
    D = table.shape[1]
    nch = B // _CH
    nwh = D // _WH

    def body(i0_ref, i1_ref, table_ref, w_hbm, b_ref, o_hbm,
             x_v, w_v, o_v, wsems, csems, osems):

        def issue_chunk(c):
            def issue_row(r, _):
                flat = i0_ref[r] * seq_len + i1_ref[r]
                pltpu.make_async_copy(
                    table_ref.at[flat], x_v.at[r], csems.at[c]
                ).start()
                return _
            lax.fori_loop(c * _CH, (c + 1) * _CH, issue_row, 0, unroll=16)

        issue_chunk(0)
        pltpu.make_async_copy(
            w_hbm.at[pl.ds(0, _WH)], w_v.at[pl.ds(0, _WH)], wsems.at[0]
        ).start()
        issue_chunk(1)
        pltpu.make_async_copy(
            w_hbm.at[pl.ds(_WH, _WH)], w_v.at[pl.ds(_WH, _WH)], wsems.at[1]
        ).start()

        for n in range(nwh):
            wsl = pl.ds(n * _WH, _WH)
            pltpu.make_async_copy(
                w_hbm.at[pl.ds(0, _WH)], w_v.at[wsl], wsems.at[n]
            ).wait()
            for c in range(nch):
                csl = pl.ds(c * _CH, _CH)
                if n == 0:
                    pltpu.make_async_copy(
                        table_ref.at[pl.ds(0, _CH)], x_v.at[csl], csems.at[c]
                    ).wait()
                acc = lax.dot_general(
                    x_v[csl, :], w_v[wsl, :],
                    (((1,), (1,)), ((), ())),
                    preferred_element_type=jnp.float32,
                )
                q = n * nch + c
                o_v[q] = jnp.tanh(acc + b_ref[:, wsl])
                pltpu.make_async_copy(
                    o_v.at[q], o_hbm.at[csl, wsl], osems.at[q]
                ).start()
        for q in range(nwh * nch):
            n, c = divmod(q, nch)
            pltpu.make_async_copy(
                o_v.at[q],
                o_hbm.at[pl.ds(c * _CH, _CH), pl.ds(n * _WH, _WH)],
                osems.at[q],
            ).wait()

    return pl.pallas_call(
        body,
        in_specs=[
            pl.BlockSpec(memory_space=pltpu.SMEM),
            pl.BlockSpec(memory_space=pltpu.SMEM),
            pl.BlockSpec(memory_space=pltpu.HBM),
            pl.BlockSpec(memory_space=pltpu.HBM),
            pl.BlockSpec(memory_space=pltpu.VMEM),
        ],
        out_specs=pl.BlockSpec(memory_space=pltpu.HBM),
        out_shape=jax.ShapeDtypeStruct((B, D), jnp.float32),
        scratch_shapes=[
            pltpu.VMEM((B, D), jnp.float32),
            pltpu.VMEM((D, D), jnp.float32),
            pltpu.VMEM((nwh * nch, _CH, _WH), jnp.float32),
            pltpu.SemaphoreType.DMA((nwh,)),
            pltpu.SemaphoreType.DMA((nch,)),
            pltpu.SemaphoreType.DMA((nwh * nch,)),
        ],
    )(idx0, idx1, table, W, b2d)


def kernel(hidden_states, cls_indexes, W, b):
    n_batch, seq_len, D = hidden_states.shape
    table = hidden_states.reshape(n_batch * seq_len, D)
    idx = cls_indexes.astype(jnp.int32)
    return _fused(table, idx[:, 0], idx[:, 1], W, b.reshape(1, D), seq_len)
